# baseline (device time: 21195 ns/iter reference)
import jax
import jax.numpy as jnp
from jax import lax
from jax.experimental import pallas as pl
from jax.experimental.pallas import tpu as pltpu


def kernel(x, dy):
    k_per, m = x.shape
    _, n = dy.shape
    m_half = m // 2
    n_slab = n // 4

    def body(x_ref, dy_ref, out_ref,
             zsend, zrecv, bcast, keep, prx, pry, prd,
             zs_sem, zr_sem, pxs, pxr, pys, pyr, pds, pdr):
        my_x = lax.axis_index("x")
        my_y = lax.axis_index("y")
        my_z = lax.axis_index("z")
        partner = (my_x, my_y, 1 - my_z)
        xn = (1 - my_x, my_y, my_z)
        yn = (my_x, 1 - my_y, my_z)
        dg = (1 - my_x, 1 - my_y, my_z)

        barrier_sem = pltpu.get_barrier_semaphore()
        for nbr in [partner, xn, yn, dg]:
            pl.semaphore_signal(
                barrier_sem, inc=1,
                device_id=nbr, device_id_type=pl.DeviceIdType.MESH,
            )
        pl.semaphore_wait(barrier_sem, 4)

        slab = 2 * my_x + my_y
        slab_x = 2 * (1 - my_x) + my_y
        slab_y = 2 * my_x + (1 - my_y)
        slab_d = 2 * (1 - my_x) + (1 - my_y)
        r_mine = my_z * m_half
        r_partner = (1 - my_z) * m_half

        dyv = dy_ref[:, pl.ds(slab * n_slab, n_slab)].astype(jnp.bfloat16)

        def piece(r0):
            xv = x_ref[:, pl.ds(r0, m_half)].astype(jnp.bfloat16)
            return lax.dot_general(
                xv, dyv, (((0,), (0,)), ((), ())),
                preferred_element_type=jnp.float32,
            )

        zsend[...] = piece(r_partner).astype(jnp.bfloat16)
        rdma_z = pltpu.make_async_remote_copy(
            src_ref=zsend, dst_ref=zrecv,
            send_sem=zs_sem, recv_sem=zr_sem,
            device_id=partner, device_id_type=pl.DeviceIdType.MESH,
        )
        rdma_z.start()

        keep[...] = piece(r_mine)

        rdma_z.wait()
        red = keep[...] + zrecv[...].astype(jnp.float32)
        out_ref[:, pl.ds(slab * n_slab, n_slab)] = red
        bcast[...] = red.astype(jnp.bfloat16)

        rdma_x = pltpu.make_async_remote_copy(
            src_ref=bcast, dst_ref=prx,
            send_sem=pxs, recv_sem=pxr,
            device_id=xn, device_id_type=pl.DeviceIdType.MESH,
        )
        rdma_y = pltpu.make_async_remote_copy(
            src_ref=bcast, dst_ref=pry,
            send_sem=pys, recv_sem=pyr,
            device_id=yn, device_id_type=pl.DeviceIdType.MESH,
        )
        rdma_d = pltpu.make_async_remote_copy(
            src_ref=bcast, dst_ref=prd,
            send_sem=pds, recv_sem=pdr,
            device_id=dg, device_id_type=pl.DeviceIdType.MESH,
        )
        rdma_x.start()
        rdma_y.start()
        rdma_d.start()

        rdma_x.wait()
        out_ref[:, pl.ds(slab_x * n_slab, n_slab)] = prx[...].astype(jnp.float32)
        rdma_y.wait()
        out_ref[:, pl.ds(slab_y * n_slab, n_slab)] = pry[...].astype(jnp.float32)
        rdma_d.wait()
        out_ref[:, pl.ds(slab_d * n_slab, n_slab)] = prd[...].astype(jnp.float32)

    return pl.pallas_call(
        body,
        out_shape=jax.ShapeDtypeStruct((m_half, n), jnp.float32),
        in_specs=[
            pl.BlockSpec(memory_space=pltpu.VMEM),
            pl.BlockSpec(memory_space=pltpu.VMEM),
        ],
        out_specs=pl.BlockSpec(memory_space=pltpu.VMEM),
        scratch_shapes=[
            pltpu.VMEM((m_half, n_slab), jnp.bfloat16),
            pltpu.VMEM((m_half, n_slab), jnp.bfloat16),
            pltpu.VMEM((m_half, n_slab), jnp.bfloat16),
            pltpu.VMEM((m_half, n_slab), jnp.float32),
            pltpu.VMEM((m_half, n_slab), jnp.bfloat16),
            pltpu.VMEM((m_half, n_slab), jnp.bfloat16),
            pltpu.VMEM((m_half, n_slab), jnp.bfloat16),
            pltpu.SemaphoreType.DMA,
            pltpu.SemaphoreType.DMA,
            pltpu.SemaphoreType.DMA,
            pltpu.SemaphoreType.DMA,
            pltpu.SemaphoreType.DMA,
            pltpu.SemaphoreType.DMA,
            pltpu.SemaphoreType.DMA,
            pltpu.SemaphoreType.DMA,
        ],
        compiler_params=pltpu.CompilerParams(collective_id=0),
    )(x, dy)
